# baseline (speedup 1.0000x reference)
import functools

import jax
import jax.numpy as jnp
from jax import lax
from jax.experimental import pallas as pl
from jax.experimental.pallas import tpu as pltpu
from jax.experimental.pallas import tpu_sc as plsc

_NUM_WORKERS = 32
_PS_ROWS = 80
_CHUNK = 64


def _make_sc_kernel(b, n_pre, n_cls, n_suf, d):
    rows = n_pre + n_cls + n_suf
    b_per_w = b // _NUM_WORKERS
    n_rounds = (b_per_w * n_cls) // _CHUNK

    mesh = plsc.VectorSubcoreMesh(core_axis_name="c", subcore_axis_name="s")

    @functools.partial(
        pl.kernel,
        mesh=mesh,
        out_type=jax.ShapeDtypeStruct((b * rows + 8, d), jnp.float32),
        scratch_types=[
            pltpu.VMEM((_PS_ROWS, d), jnp.float32),
            pltpu.VMEM((_CHUNK, d), jnp.float32),
            pltpu.VMEM((n_rounds, _CHUNK), jnp.int32),
            pltpu.VMEM((n_rounds, _CHUNK), jnp.int32),
            pltpu.VMEM((b_per_w, _PS_ROWS), jnp.int32),
            pltpu.SemaphoreType.DMA,
            pltpu.SemaphoreType.DMA,
        ],
    )
    def sck(cls_h, ps_h, cidx_h, oidx_h, psidx_h, out_h,
            ps_v, g_v, cidx_v, oidx_v, psidx_v, sem_g, sem_l):
        wid = lax.axis_index("s") * 2 + lax.axis_index("c")
        pltpu.sync_copy(ps_h, ps_v)
        pltpu.sync_copy(cidx_h.at[wid], cidx_v)
        pltpu.sync_copy(oidx_h.at[wid], oidx_v)
        pltpu.sync_copy(psidx_h.at[wid], psidx_v)

        def group_body(k, carry):
            pltpu.async_copy(cls_h.at[cidx_v.at[k]], g_v, sem_g).wait()
            pltpu.async_copy(g_v, out_h.at[oidx_v.at[k]], sem_g).wait()
            return carry

        lax.fori_loop(0, n_rounds, group_body, 0)

        def lin_body(t, carry):
            descs = []
            for j in range(8):
                descs.append(
                    pltpu.async_copy(
                        ps_v, out_h.at[psidx_v.at[t * 8 + j]], sem_l))
            for dd in descs:
                dd.wait()
            return carry

        lax.fori_loop(0, b_per_w // 8, lin_body, 0)

    return sck


def kernel(label, cls_ctx, token_prefix, token_suffix):
    b = label.shape[0]
    num_class, n_cls, d = cls_ctx.shape
    n_pre = token_prefix.shape[1]
    n_suf = token_suffix.shape[1]
    rows = n_pre + n_cls + n_suf
    n_ps = n_pre + n_suf
    dump = b * rows

    lab = label.astype(jnp.int32)
    cls_tab = cls_ctx.reshape(num_class * n_cls, d)
    ps = jnp.concatenate(
        [token_prefix[0], token_suffix[0],
         jnp.zeros((_PS_ROWS - n_ps, d), jnp.float32)], axis=0)

    cidx = (lab[:, None] * n_cls + jnp.arange(n_cls, dtype=jnp.int32)
            ).reshape(_NUM_WORKERS, -1, _CHUNK)
    orow = ((jnp.arange(b, dtype=jnp.int32) * rows + n_pre)[:, None]
            + jnp.arange(n_cls, dtype=jnp.int32)
            ).reshape(_NUM_WORKERS, -1, _CHUNK)

    i = jnp.arange(_PS_ROWS, dtype=jnp.int32)
    off = jnp.where(i < n_pre, i, i + n_cls)
    psidx = jnp.where(
        i[None, :] < n_ps,
        jnp.arange(b, dtype=jnp.int32)[:, None] * rows + off[None, :],
        dump + jnp.maximum(i[None, :] - n_ps, 0),
    ).reshape(_NUM_WORKERS, -1, _PS_ROWS)

    sck = _make_sc_kernel(b, n_pre, n_cls, n_suf, d)
    out = sck(cls_tab, ps, cidx, orow, psidx)
    return out[:b * rows].reshape(b, rows, d)
